# Initial kernel scaffold; baseline (speedup 1.0000x reference)
import jax
import jax.numpy as jnp
from jax.experimental import pallas as pl


def kernel(X, E_in_idx, E_ex_idx, W_node, b_node, W_edge, b_edge, gain_nodes, bias_nodes, gain_edges, bias_edges):
    raise NotImplementedError("write your pallas kernel here")


# speedup vs baseline: 17.0873x; 17.0873x over previous
import functools

import jax
import jax.numpy as jnp
from jax import lax
from jax.experimental import pallas as pl
from jax.experimental.pallas import tpu as pltpu
from jax.experimental.pallas import tpu_sc as plsc

N_NODES = 10000
N_EDGES = 160000
NUM_RBF = 16
RBF_SIGMA = 20.0 / NUM_RBF
RBF_STEP = 20.0 / (NUM_RBF - 1)

_ATOM_OFF = {"N": 0, "Ca": 3, "C": 6, "O": 9}
_EDGE_LIST = ["Ca-Ca", "Ca-C", "C-Ca", "Ca-N", "N-Ca", "Ca-O", "O-Ca",
              "C-C", "C-N", "N-C", "C-O", "O-C", "N-N", "N-O", "O-N", "O-O"]
_NODE_LIST = ["Ca-N", "Ca-C", "Ca-O", "N-C", "N-O", "O-C"]

_NW = 32
_CHUNK = 2000

_BE = 2000
_BN = 1000


def _layernorm(h, gain, bias, inv_dof):
    mu = jnp.mean(h, axis=1, keepdims=True)
    xc = h - mu
    var = jnp.sum(xc * xc, axis=1, keepdims=True) * inv_dof
    sigma = jnp.sqrt(var + 1e-6)
    return gain * xc / (sigma + 1e-6) + bias


def _sc_gather(table, idx):
    total = idx.shape[0]
    per_w = total // _NW
    steps = per_w // _CHUNK
    mesh = plsc.VectorSubcoreMesh(core_axis_name="c", subcore_axis_name="s")

    @functools.partial(
        pl.kernel,
        out_type=jax.ShapeDtypeStruct((total, 16), jnp.float32),
        mesh=mesh,
        scratch_types=[
            pltpu.VMEM((_CHUNK,), jnp.int32),
            pltpu.VMEM((_CHUNK, 16), jnp.float32),
            pltpu.SemaphoreType.DMA,
        ],
        compiler_params=pltpu.CompilerParams(use_tc_tiling_on_sc=False),
    )
    def k(table_hbm, idx_hbm, out_hbm, idx_v, rows_v, sem):
        wid = lax.axis_index("s") * 2 + lax.axis_index("c")
        base = wid * per_w
        for i in range(steps):
            off = base + i * _CHUNK
            pltpu.sync_copy(idx_hbm.at[pl.ds(off, _CHUNK)], idx_v)
            pltpu.async_copy(table_hbm.at[idx_v], rows_v, sem).wait()
            pltpu.sync_copy(rows_v, out_hbm.at[pl.ds(off, _CHUNK)])

    return k(table, idx)


def _pairs(pair_list):
    return [(_ATOM_OFF[p.split("-")[0]] // 3, _ATOM_OFF[p.split("-")[1]] // 3)
            for p in pair_list]


def _rbf_featT(xaT, xbT, pairs, eps):
    npair = len(pairs)
    d2 = None
    for c in range(3):
        sa = jnp.concatenate([xaT[3 * a + c:3 * a + c + 1] for a, _ in pairs],
                             axis=0)
        sb = jnp.concatenate([xbT[3 * b + c:3 * b + c + 1] for _, b in pairs],
                             axis=0)
        dd = sa - sb
        d2 = dd * dd if d2 is None else d2 + dd * dd
    dist = jnp.sqrt(d2 + eps) if eps else jnp.sqrt(d2)
    cols = dist.shape[1]
    dist_r = jnp.concatenate(
        [jnp.broadcast_to(dist[p:p + 1], (NUM_RBF, cols))
         for p in range(npair)], axis=0)
    mu = lax.broadcasted_iota(jnp.int32, (NUM_RBF, 1), 0).astype(jnp.float32)
    mu_r = jnp.concatenate([mu] * npair, axis=0) * RBF_STEP
    z = (dist_r - mu_r) / RBF_SIGMA
    return jnp.exp(-(z * z))


def _nrmT(v):
    n = jnp.sqrt(v[0:1] * v[0:1] + v[1:2] * v[1:2] + v[2:3] * v[2:3])
    n = jnp.where(n == 0.0, 1.0, n)
    return v / n


def _crossT(u, v):
    return jnp.concatenate([
        u[1:2] * v[2:3] - u[2:3] * v[1:2],
        u[2:3] * v[0:1] - u[0:1] * v[2:3],
        u[0:1] * v[1:2] - u[1:2] * v[0:1]], axis=0)


def _edge_body(gs_ref, gd_ref, w_ref, b_ref, gain_ref, bias_ref, o_ref):
    xs_t = gs_ref[:, :13].T
    xd_t = gd_ref[:, :12].T

    rbf = _rbf_featT(xs_t, xd_t, _pairs(_EDGE_LIST), 1e-6)

    n_s = xs_t[0:3]
    qv = xs_t[12:13]
    u0 = _nrmT(xs_t[3:6] - n_s)
    u1 = _nrmT(xs_t[6:9] - xs_t[3:6])
    n0 = _nrmT(_crossT(u0, u1))
    b1 = _nrmT(u0 - u1)
    c2 = _crossT(b1, n0)
    dirs = []
    for ob in (3, 0, 6, 9):
        v = xd_t[ob:ob + 3] - n_s
        du = (b1 * v[0:1] + n0 * v[1:2] + c2 * v[2:3]) * qv
        dirs.append(_nrmT(du))

    f = jnp.concatenate([rbf] + dirs, axis=0)
    h = lax.dot_general(f, w_ref[...], (((0,), (0,)), ((), ())),
                        preferred_element_type=jnp.float32) + b_ref[...]
    o_ref[...] = _layernorm(h, gain_ref[...], bias_ref[...], 1.0 / 255.0)


def _edge_call(g, w, b, gain, bias, src_blk, dst_blk):
    nblk = N_EDGES // _BE
    return pl.pallas_call(
        _edge_body,
        grid=(nblk,),
        in_specs=[
            pl.BlockSpec((_BE, 16), lambda i, o=src_blk: (i + o, 0)),
            pl.BlockSpec((_BE, 16), lambda i, o=dst_blk: (i + o, 0)),
            pl.BlockSpec(w.shape, lambda i: (0, 0)),
            pl.BlockSpec(b.shape, lambda i: (0, 0)),
            pl.BlockSpec(gain.shape, lambda i: (0, 0)),
            pl.BlockSpec(bias.shape, lambda i: (0, 0)),
        ],
        out_specs=pl.BlockSpec((_BE, w.shape[1]), lambda i: (i, 0)),
        out_shape=jax.ShapeDtypeStruct((N_EDGES, w.shape[1]), jnp.float32),
    )(g, g, w, b, gain, bias)


def _node_body(x_ref, w_ref, b_ref, gain_ref, bias_ref, o_ref):
    x_t = x_ref[...].T
    f = _rbf_featT(x_t, x_t, _pairs(_NODE_LIST), 0.0)
    h = lax.dot_general(f, w_ref[...], (((0,), (0,)), ((), ())),
                        preferred_element_type=jnp.float32) + b_ref[...]
    o_ref[...] = _layernorm(h, gain_ref[...], bias_ref[...], 1.0 / 127.0)


def _node_call(x12, w, b, gain, bias):
    nblk = N_NODES // _BN
    return pl.pallas_call(
        _node_body,
        grid=(nblk,),
        in_specs=[
            pl.BlockSpec((_BN, 12), lambda i: (i, 0)),
            pl.BlockSpec(w.shape, lambda i: (0, 0)),
            pl.BlockSpec(b.shape, lambda i: (0, 0)),
            pl.BlockSpec(gain.shape, lambda i: (0, 0)),
            pl.BlockSpec(bias.shape, lambda i: (0, 0)),
        ],
        out_specs=pl.BlockSpec((_BN, w.shape[1]), lambda i: (i, 0)),
        out_shape=jax.ShapeDtypeStruct((N_NODES, w.shape[1]), jnp.float32),
    )(x12, w, b, gain, bias)


def kernel(X, E_in_idx, E_ex_idx, W_node, b_node, W_edge, b_edge,
           gain_nodes, bias_nodes, gain_edges, bias_edges):
    n = X.shape[0]
    e = E_in_idx.shape[1]
    x12 = X.reshape(n, 12)
    qvalid = (jnp.arange(n, dtype=jnp.int32) < n - 1).astype(jnp.float32)
    table = jnp.concatenate(
        [x12, qvalid[:, None], jnp.zeros((n, 3), jnp.float32)], axis=1)
    idx = jnp.concatenate([E_in_idx.reshape(-1), E_ex_idx.reshape(-1)])

    g = _sc_gather(table, idx)

    b_n = b_node.reshape(1, -1)
    g_n = gain_nodes.reshape(1, -1)
    bi_n = bias_nodes.reshape(1, -1)
    b_e = b_edge.reshape(1, -1)
    g_e = gain_edges.reshape(1, -1)
    bi_e = bias_edges.reshape(1, -1)

    h_v = _node_call(x12, W_node, b_n, g_n, bi_n)
    eb = e // _BE
    h_in = _edge_call(g, W_edge, b_e, g_e, bi_e, 0, eb)
    h_ex = _edge_call(g, W_edge, b_e, g_e, bi_e, 2 * eb, 3 * eb)
    return (h_v, h_in, h_ex)


# speedup vs baseline: 17.4183x; 1.0194x over previous
import functools

import jax
import jax.numpy as jnp
from jax import lax
from jax.experimental import pallas as pl
from jax.experimental.pallas import tpu as pltpu
from jax.experimental.pallas import tpu_sc as plsc

N_NODES = 10000
N_EDGES = 160000
NUM_RBF = 16
RBF_SIGMA = 20.0 / NUM_RBF
RBF_STEP = 20.0 / (NUM_RBF - 1)

_ATOM_OFF = {"N": 0, "Ca": 3, "C": 6, "O": 9}
_EDGE_LIST = ["Ca-Ca", "Ca-C", "C-Ca", "Ca-N", "N-Ca", "Ca-O", "O-Ca",
              "C-C", "C-N", "N-C", "C-O", "O-C", "N-N", "N-O", "O-N", "O-O"]
_NODE_LIST = ["Ca-N", "Ca-C", "Ca-O", "N-C", "N-O", "O-C"]

_NW = 32
_CHUNK = 2000

_BE = 4000
_BN = 1000


def _layernorm(h, gain, bias, inv_dof):
    mu = jnp.mean(h, axis=1, keepdims=True)
    xc = h - mu
    var = jnp.sum(xc * xc, axis=1, keepdims=True) * inv_dof
    sigma = jnp.sqrt(var + 1e-6)
    return gain * xc / (sigma + 1e-6) + bias


def _sc_gather(table, idx):
    total = idx.shape[0]
    per_w = total // _NW
    steps = per_w // _CHUNK
    mesh = plsc.VectorSubcoreMesh(core_axis_name="c", subcore_axis_name="s")

    @functools.partial(
        pl.kernel,
        out_type=jax.ShapeDtypeStruct((total, 16), jnp.float32),
        mesh=mesh,
        scratch_types=[
            pltpu.VMEM((_CHUNK,), jnp.int32),
            pltpu.VMEM((_CHUNK,), jnp.int32),
            pltpu.VMEM((_CHUNK, 16), jnp.float32),
            pltpu.VMEM((_CHUNK, 16), jnp.float32),
            pltpu.SemaphoreType.DMA,
            pltpu.SemaphoreType.DMA,
            pltpu.SemaphoreType.DMA,
            pltpu.SemaphoreType.DMA,
        ],
        compiler_params=pltpu.CompilerParams(use_tc_tiling_on_sc=False),
    )
    def k(table_hbm, idx_hbm, out_hbm, i0, i1, r0, r1, g0, g1, o0, o1):
        wid = lax.axis_index("s") * 2 + lax.axis_index("c")
        base = wid * per_w
        ib = [i0, i1]
        rb = [r0, r1]
        gsem = [g0, g1]
        osem = [o0, o1]
        pend_g = [None, None]
        pend_o = [None, None]

        pltpu.sync_copy(idx_hbm.at[pl.ds(base, _CHUNK)], ib[0])
        pend_g[0] = pltpu.make_async_copy(table_hbm.at[ib[0]], rb[0], gsem[0])
        pend_g[0].start()

        for i in range(steps):
            s = i % 2
            ns = 1 - s
            if i + 1 < steps:
                off = base + (i + 1) * _CHUNK
                pltpu.sync_copy(idx_hbm.at[pl.ds(off, _CHUNK)], ib[ns])
                if pend_o[ns] is not None:
                    pend_o[ns].wait()
                    pend_o[ns] = None
                pend_g[ns] = pltpu.make_async_copy(
                    table_hbm.at[ib[ns]], rb[ns], gsem[ns])
                pend_g[ns].start()
            pend_g[s].wait()
            off = base + i * _CHUNK
            pend_o[s] = pltpu.make_async_copy(
                rb[s], out_hbm.at[pl.ds(off, _CHUNK)], osem[s])
            pend_o[s].start()
        for s in range(2):
            if pend_o[s] is not None:
                pend_o[s].wait()

    return k(table, idx)


def _pairs(pair_list):
    return [(_ATOM_OFF[p.split("-")[0]] // 3, _ATOM_OFF[p.split("-")[1]] // 3)
            for p in pair_list]


def _rbf_featT(xaT, xbT, pairs, eps):
    npair = len(pairs)
    d2 = None
    for c in range(3):
        sa = jnp.concatenate([xaT[3 * a + c:3 * a + c + 1] for a, _ in pairs],
                             axis=0)
        sb = jnp.concatenate([xbT[3 * b + c:3 * b + c + 1] for _, b in pairs],
                             axis=0)
        dd = sa - sb
        d2 = dd * dd if d2 is None else d2 + dd * dd
    dist = jnp.sqrt(d2 + eps) if eps else jnp.sqrt(d2)
    cols = dist.shape[1]
    dist_r = jnp.concatenate(
        [jnp.broadcast_to(dist[p:p + 1], (NUM_RBF, cols))
         for p in range(npair)], axis=0)
    mu = lax.broadcasted_iota(jnp.int32, (NUM_RBF, 1), 0).astype(jnp.float32)
    mu_r = jnp.concatenate([mu] * npair, axis=0) * RBF_STEP
    z = (dist_r - mu_r) / RBF_SIGMA
    return jnp.exp(-(z * z))


def _nrmT(v):
    n = jnp.sqrt(v[0:1] * v[0:1] + v[1:2] * v[1:2] + v[2:3] * v[2:3])
    n = jnp.where(n == 0.0, 1.0, n)
    return v / n


def _crossT(u, v):
    return jnp.concatenate([
        u[1:2] * v[2:3] - u[2:3] * v[1:2],
        u[2:3] * v[0:1] - u[0:1] * v[2:3],
        u[0:1] * v[1:2] - u[1:2] * v[0:1]], axis=0)


def _edge_body(gs_ref, gd_ref, w_ref, b_ref, gain_ref, bias_ref, o_ref):
    xs_t = gs_ref[:, :13].T
    xd_t = gd_ref[:, :12].T

    rbf = _rbf_featT(xs_t, xd_t, _pairs(_EDGE_LIST), 1e-6)

    n_s = xs_t[0:3]
    qv = xs_t[12:13]
    u0 = _nrmT(xs_t[3:6] - n_s)
    u1 = _nrmT(xs_t[6:9] - xs_t[3:6])
    n0 = _nrmT(_crossT(u0, u1))
    b1 = _nrmT(u0 - u1)
    c2 = _crossT(b1, n0)
    dirs = []
    for ob in (3, 0, 6, 9):
        v = xd_t[ob:ob + 3] - n_s
        du = (b1 * v[0:1] + n0 * v[1:2] + c2 * v[2:3]) * qv
        dirs.append(_nrmT(du))

    f = jnp.concatenate([rbf] + dirs, axis=0)
    h = lax.dot_general(f, w_ref[...], (((0,), (0,)), ((), ())),
                        preferred_element_type=jnp.float32) + b_ref[...]
    o_ref[...] = _layernorm(h, gain_ref[...], bias_ref[...], 1.0 / 255.0)


def _edge_call(g, w, b, gain, bias, src_blk, dst_blk):
    nblk = N_EDGES // _BE
    return pl.pallas_call(
        _edge_body,
        grid=(nblk,),
        in_specs=[
            pl.BlockSpec((_BE, 16), lambda i, o=src_blk: (i + o, 0)),
            pl.BlockSpec((_BE, 16), lambda i, o=dst_blk: (i + o, 0)),
            pl.BlockSpec(w.shape, lambda i: (0, 0)),
            pl.BlockSpec(b.shape, lambda i: (0, 0)),
            pl.BlockSpec(gain.shape, lambda i: (0, 0)),
            pl.BlockSpec(bias.shape, lambda i: (0, 0)),
        ],
        out_specs=pl.BlockSpec((_BE, w.shape[1]), lambda i: (i, 0)),
        out_shape=jax.ShapeDtypeStruct((N_EDGES, w.shape[1]), jnp.float32),
    )(g, g, w, b, gain, bias)


def _node_body(x_ref, w_ref, b_ref, gain_ref, bias_ref, o_ref):
    x_t = x_ref[...].T
    f = _rbf_featT(x_t, x_t, _pairs(_NODE_LIST), 0.0)
    h = lax.dot_general(f, w_ref[...], (((0,), (0,)), ((), ())),
                        preferred_element_type=jnp.float32) + b_ref[...]
    o_ref[...] = _layernorm(h, gain_ref[...], bias_ref[...], 1.0 / 127.0)


def _node_call(x12, w, b, gain, bias):
    nblk = N_NODES // _BN
    return pl.pallas_call(
        _node_body,
        grid=(nblk,),
        in_specs=[
            pl.BlockSpec((_BN, 12), lambda i: (i, 0)),
            pl.BlockSpec(w.shape, lambda i: (0, 0)),
            pl.BlockSpec(b.shape, lambda i: (0, 0)),
            pl.BlockSpec(gain.shape, lambda i: (0, 0)),
            pl.BlockSpec(bias.shape, lambda i: (0, 0)),
        ],
        out_specs=pl.BlockSpec((_BN, w.shape[1]), lambda i: (i, 0)),
        out_shape=jax.ShapeDtypeStruct((N_NODES, w.shape[1]), jnp.float32),
    )(x12, w, b, gain, bias)


def kernel(X, E_in_idx, E_ex_idx, W_node, b_node, W_edge, b_edge,
           gain_nodes, bias_nodes, gain_edges, bias_edges):
    n = X.shape[0]
    e = E_in_idx.shape[1]
    x12 = X.reshape(n, 12)
    qvalid = (jnp.arange(n, dtype=jnp.int32) < n - 1).astype(jnp.float32)
    table = jnp.concatenate(
        [x12, qvalid[:, None], jnp.zeros((n, 3), jnp.float32)], axis=1)
    idx = jnp.concatenate([E_in_idx.reshape(-1), E_ex_idx.reshape(-1)])

    g = _sc_gather(table, idx)

    b_n = b_node.reshape(1, -1)
    g_n = gain_nodes.reshape(1, -1)
    bi_n = bias_nodes.reshape(1, -1)
    b_e = b_edge.reshape(1, -1)
    g_e = gain_edges.reshape(1, -1)
    bi_e = bias_edges.reshape(1, -1)

    h_v = _node_call(x12, W_node, b_n, g_n, bi_n)
    eb = e // _BE
    h_in = _edge_call(g, W_edge, b_e, g_e, bi_e, 0, eb)
    h_ex = _edge_call(g, W_edge, b_e, g_e, bi_e, 2 * eb, 3 * eb)
    return (h_v, h_in, h_ex)
